# Initial kernel scaffold; baseline (speedup 1.0000x reference)
import jax
import jax.numpy as jnp
from jax.experimental import pallas as pl


def kernel(x, emb):
    raise NotImplementedError("write your pallas kernel here")


# speedup vs baseline: 26.0757x; 26.0757x over previous
import functools

import jax
import jax.numpy as jnp
from jax.experimental import pallas as pl

_N_CODES = 1024
_DIM = 64
_BC = 2048
_N_ELEMS = 256 * 256
_GRID = _N_ELEMS // _BC


def _vq_body(x_ref, emb_ref, q_ref, loss_ref):
    i = pl.program_id(0)
    t = x_ref[0]
    emb = emb_ref[...]

    p = t
    while p.shape[0] < _DIM:
        m = p.shape[0]
        p = jnp.concatenate([p, p * p[m - 1:m]], axis=0)

    e2 = jnp.sum(emb * emb, axis=1, keepdims=True)
    dots = jax.lax.dot_general(
        emb, p, (((1,), (0,)), ((), ())),
        preferred_element_type=jnp.float32)
    d = e2 - 2.0 * dots

    mval = jnp.min(d, axis=0, keepdims=True)
    jidx = jax.lax.broadcasted_iota(jnp.int32, d.shape, 0)
    idx = jnp.min(jnp.where(d == mval, jidx, _N_CODES), axis=0,
                  keepdims=True)

    idxf = idx.astype(jnp.float32)
    q_ref[0] = t + (idxf - t)

    p2 = jnp.sum(p * p, axis=0, keepdims=True)
    qpart = jnp.sum(mval + p2) / (_N_ELEMS * _DIM)
    epart = jnp.sum((t - idxf) ** 2) / _N_ELEMS
    contrib = jnp.full((1, 1), 0.0, jnp.float32) + qpart + 0.25 * epart

    @pl.when(i == 0)
    def _():
        loss_ref[...] = jnp.zeros((1, 1), jnp.float32)

    loss_ref[...] += contrib


@functools.partial(jax.jit, static_argnames=("interpret",))
def kernel(x, emb, interpret=False):
    xf = x.reshape(_GRID, 1, _BC)
    q, loss = pl.pallas_call(
        _vq_body,
        grid=(_GRID,),
        in_specs=[
            pl.BlockSpec((1, 1, _BC), lambda i: (i, 0, 0)),
            pl.BlockSpec((_N_CODES, _DIM), lambda i: (0, 0)),
        ],
        out_specs=[
            pl.BlockSpec((1, 1, _BC), lambda i: (i, 0, 0)),
            pl.BlockSpec((1, 1), lambda i: (0, 0)),
        ],
        out_shape=[
            jax.ShapeDtypeStruct((_GRID, 1, _BC), jnp.float32),
            jax.ShapeDtypeStruct((1, 1), jnp.float32),
        ],
        interpret=interpret,
    )(xf, emb)
    return q.reshape(x.shape), loss[0, 0]


# speedup vs baseline: 32.2509x; 1.2368x over previous
import functools

import jax
import jax.numpy as jnp
from jax.experimental import pallas as pl

_N_CODES = 1024
_DIM = 64
_BC = 8192
_JC = 128
_N_ELEMS = 256 * 256
_GRID = _N_ELEMS // _BC


def _vq_body(x_ref, emb_ref, q_ref, loss_ref):
    i = pl.program_id(0)
    t = x_ref[0]
    emb = emb_ref[...]

    p = t
    while p.shape[0] < _DIM:
        m = p.shape[0]
        p = jnp.concatenate([p, p * p[m - 1:m]], axis=0)

    e2 = jnp.sum(emb * emb, axis=1, keepdims=True)
    nemb = -2.0 * emb
    jl = jax.lax.broadcasted_iota(
        jnp.int32, (_JC, _BC), 0).astype(jnp.float32)

    mval = None
    for c in range(_N_CODES // _JC):
        dc = jax.lax.dot_general(
            nemb[c * _JC:(c + 1) * _JC], p, (((1,), (0,)), ((), ())),
            preferred_element_type=jnp.float32) + e2[c * _JC:(c + 1) * _JC]
        mc = jnp.min(dc, axis=0, keepdims=True)
        ic = jnp.min(jnp.where(dc == mc, jl, 3e38), axis=0,
                     keepdims=True) + (c * _JC)
        if mval is None:
            mval, idxf = mc, ic
        else:
            better = mc < mval
            mval = jnp.where(better, mc, mval)
            idxf = jnp.where(better, ic, idxf)

    q_ref[0] = t + (idxf - t)

    p2 = jnp.sum(p * p, axis=0, keepdims=True)
    qpart = jnp.sum(mval + p2) / (_N_ELEMS * _DIM)
    epart = jnp.sum((t - idxf) ** 2) / _N_ELEMS
    contrib = jnp.full((1, 1), 0.0, jnp.float32) + qpart + 0.25 * epart

    @pl.when(i == 0)
    def _():
        loss_ref[...] = jnp.zeros((1, 1), jnp.float32)

    loss_ref[...] += contrib


@functools.partial(jax.jit, static_argnames=("interpret",))
def kernel(x, emb, interpret=False):
    xf = x.reshape(_GRID, 1, _BC)
    q, loss = pl.pallas_call(
        _vq_body,
        grid=(_GRID,),
        in_specs=[
            pl.BlockSpec((1, 1, _BC), lambda i: (i, 0, 0)),
            pl.BlockSpec((_N_CODES, _DIM), lambda i: (0, 0)),
        ],
        out_specs=[
            pl.BlockSpec((1, 1, _BC), lambda i: (i, 0, 0)),
            pl.BlockSpec((1, 1), lambda i: (0, 0)),
        ],
        out_shape=[
            jax.ShapeDtypeStruct((_GRID, 1, _BC), jnp.float32),
            jax.ShapeDtypeStruct((1, 1), jnp.float32),
        ],
        interpret=interpret,
    )(xf, emb)
    return q.reshape(x.shape), loss[0, 0]


# speedup vs baseline: 40.3506x; 1.2511x over previous
import functools

import jax
import jax.numpy as jnp
from jax.experimental import pallas as pl

_N_CODES = 1024
_DIM = 64
_BC = 8192
_JC = 128
_N_ELEMS = 256 * 256
_GRID = _N_ELEMS // _BC


def _vq_body(x_ref, emb_ref, q_ref, loss_ref):
    i = pl.program_id(0)
    t = x_ref[0]
    emb = emb_ref[...]

    p = t
    while p.shape[0] < _DIM:
        m = p.shape[0]
        p = jnp.concatenate([p, p * p[m - 1:m]], axis=0)

    e2 = jnp.sum(emb * emb, axis=1, keepdims=True)
    nemb = -2.0 * emb
    jl = jax.lax.broadcasted_iota(
        jnp.int32, (_JC, _BC), 0).astype(jnp.float32)

    mval = wchunk = seldc = None
    for c in range(_N_CODES // _JC):
        dc = jax.lax.dot_general(
            nemb[c * _JC:(c + 1) * _JC], p, (((1,), (0,)), ((), ())),
            preferred_element_type=jnp.float32) + e2[c * _JC:(c + 1) * _JC]
        mc = jnp.min(dc, axis=0, keepdims=True)
        if mval is None:
            mval, wchunk, seldc = mc, jnp.zeros_like(mc), dc
        else:
            better = mc < mval
            mval = jnp.where(better, mc, mval)
            wchunk = jnp.where(better, float(c), wchunk)
            seldc = jnp.where(better, dc, seldc)

    ic = jnp.min(jnp.where(seldc == mval, jl, 3e38), axis=0, keepdims=True)
    idxf = ic + wchunk * float(_JC)

    q_ref[0] = t + (idxf - t)

    p2 = jnp.sum(p * p, axis=0, keepdims=True)
    qpart = jnp.sum(mval + p2) / (_N_ELEMS * _DIM)
    epart = jnp.sum((t - idxf) ** 2) / _N_ELEMS
    contrib = jnp.full((1, 1), 0.0, jnp.float32) + qpart + 0.25 * epart

    @pl.when(i == 0)
    def _():
        loss_ref[...] = jnp.zeros((1, 1), jnp.float32)

    loss_ref[...] += contrib


@functools.partial(jax.jit, static_argnames=("interpret",))
def kernel(x, emb, interpret=False):
    xf = x.reshape(_GRID, 1, _BC)
    q, loss = pl.pallas_call(
        _vq_body,
        grid=(_GRID,),
        in_specs=[
            pl.BlockSpec((1, 1, _BC), lambda i: (i, 0, 0)),
            pl.BlockSpec((_N_CODES, _DIM), lambda i: (0, 0)),
        ],
        out_specs=[
            pl.BlockSpec((1, 1, _BC), lambda i: (i, 0, 0)),
            pl.BlockSpec((1, 1), lambda i: (0, 0)),
        ],
        out_shape=[
            jax.ShapeDtypeStruct((_GRID, 1, _BC), jnp.float32),
            jax.ShapeDtypeStruct((1, 1), jnp.float32),
        ],
        interpret=interpret,
    )(xf, emb)
    return q.reshape(x.shape), loss[0, 0]


# speedup vs baseline: 41.5851x; 1.0306x over previous
import functools

import jax
import jax.numpy as jnp
from jax.experimental import pallas as pl

_N_CODES = 1024
_DIM = 64
_BC = 8192
_JC = 128
_N_ELEMS = 256 * 256
_GRID = _N_ELEMS // _BC


def _vq_body(x_ref, emb_ref, q_ref, loss_ref):
    i = pl.program_id(0)
    t = x_ref[0]
    emb = emb_ref[...]

    p = t
    while p.shape[0] < _DIM:
        m = p.shape[0]
        p = jnp.concatenate([p, p * p[m - 1:m]], axis=0)

    e2 = jnp.sum(emb * emb, axis=1, keepdims=True)
    b1 = (e2.astype(jnp.bfloat16)).astype(jnp.float32)
    r1 = e2 - b1
    b2 = (r1.astype(jnp.bfloat16)).astype(jnp.float32)
    b3 = r1 - b2
    nemb = jnp.concatenate(
        [-2.0 * emb, b1, b2, b3, jnp.zeros((_N_CODES, 61), jnp.float32)],
        axis=1)
    pa = jnp.concatenate(
        [p, jnp.full((3, _BC), 1.0, jnp.float32),
         jnp.zeros((61, _BC), jnp.float32)], axis=0)
    jl = jax.lax.broadcasted_iota(
        jnp.int32, (_JC, _BC), 0).astype(jnp.float32)

    mval = wchunk = seldc = None
    for c in range(_N_CODES // _JC):
        dc = jax.lax.dot_general(
            nemb[c * _JC:(c + 1) * _JC], pa, (((1,), (0,)), ((), ())),
            preferred_element_type=jnp.float32)
        mc = jnp.min(dc, axis=0, keepdims=True)
        if mval is None:
            mval, wchunk, seldc = mc, jnp.zeros_like(mc), dc
        else:
            better = mc < mval
            mval = jnp.where(better, mc, mval)
            wchunk = jnp.where(better, float(c), wchunk)
            seldc = jnp.where(better, dc, seldc)

    ic = jnp.min(jnp.where(seldc == mval, jl, 3e38), axis=0, keepdims=True)
    idxf = ic + wchunk * float(_JC)

    q_ref[0] = t + (idxf - t)

    p2 = jnp.sum(p * p, axis=0, keepdims=True)
    qpart = jnp.sum(mval + p2) / (_N_ELEMS * _DIM)
    epart = jnp.sum((t - idxf) ** 2) / _N_ELEMS
    contrib = jnp.full((1, 1), 0.0, jnp.float32) + qpart + 0.25 * epart

    @pl.when(i == 0)
    def _():
        loss_ref[...] = jnp.zeros((1, 1), jnp.float32)

    loss_ref[...] += contrib


@functools.partial(jax.jit, static_argnames=("interpret",))
def kernel(x, emb, interpret=False):
    xf = x.reshape(_GRID, 1, _BC)
    q, loss = pl.pallas_call(
        _vq_body,
        grid=(_GRID,),
        in_specs=[
            pl.BlockSpec((1, 1, _BC), lambda i: (i, 0, 0)),
            pl.BlockSpec((_N_CODES, _DIM), lambda i: (0, 0)),
        ],
        out_specs=[
            pl.BlockSpec((1, 1, _BC), lambda i: (i, 0, 0)),
            pl.BlockSpec((1, 1), lambda i: (0, 0)),
        ],
        out_shape=[
            jax.ShapeDtypeStruct((_GRID, 1, _BC), jnp.float32),
            jax.ShapeDtypeStruct((1, 1), jnp.float32),
        ],
        interpret=interpret,
    )(xf, emb)
    return q.reshape(x.shape), loss[0, 0]
